# baseline (speedup 1.0000x reference)
import functools

import jax
import jax.numpy as jnp
from jax import lax
from jax.experimental import pallas as pl
from jax.experimental.pallas import tpu as pltpu
from jax.experimental.pallas import tpu_sc as plsc

NC = 2
NS = 16
NW = NC * NS
LANES = 16

B = 128
DR = 80
TR = 640


def _sc_mesh():
    return plsc.VectorSubcoreMesh(core_axis_name="c", subcore_axis_name="s")


DW = 128


def _degree_partials(dst, ew, n):
    e = dst.shape[0]
    nblk = e // B

    @functools.partial(
        pl.kernel,
        out_type=jax.ShapeDtypeStruct((NC, n, DW), jnp.float32),
        mesh=_sc_mesh(),
        scratch_types=[
            pltpu.VMEM((B,), jnp.int32),
            pltpu.VMEM((B,), jnp.float32),
            pltpu.VMEM((B, DW), jnp.float32),
            pltpu.VMEM_SHARED((n, DW), jnp.float32),
        ],
    )
    def k(dst_hbm, ew_hbm, out_hbm, idx_v, ew_v, rows_v, acc_sh):
        c = lax.axis_index("c")
        s = lax.axis_index("s")
        wid = s * NC + c

        zero16 = jnp.zeros((LANES,), jnp.float32)

        def zrow(i, _):
            for j in range(DW // LANES):
                rows_v[i, pl.ds(j * LANES, LANES)] = zero16
            return 0

        lax.fori_loop(0, B, zrow, 0)

        r0 = s * TR
        nch = jnp.where(s < NS - 1, TR // DR, (n - (NS - 1) * TR) // DR)

        def zacc(i, _):
            pltpu.sync_copy(rows_v.at[pl.ds(0, DR)],
                            acc_sh.at[pl.ds(r0 + i * DR, DR)])
            return 0

        lax.fori_loop(0, nch, zacc, 0)
        plsc.subcore_barrier()

        nblk_w = (nblk // NW) + jnp.where(wid < (nblk % NW), 1, 0)

        def blk(b, _):
            off = (wid + b * NW) * B
            pltpu.sync_copy(dst_hbm.at[pl.ds(off, B)], idx_v)
            pltpu.sync_copy(ew_hbm.at[pl.ds(off, B)], ew_v)

            def sgrp(g, _):
                sv16 = ew_v[pl.ds(g * LANES, LANES)]
                for r in range(LANES):
                    sv = jnp.broadcast_to(sv16[r], (LANES,))
                    for j in range(DW // LANES):
                        rows_v[g * LANES + r, pl.ds(j * LANES, LANES)] = sv
                return 0

            lax.fori_loop(0, B // LANES, sgrp, 0)
            pltpu.sync_copy(rows_v, acc_sh.at[idx_v], add=True)
            return 0

        lax.fori_loop(0, nblk_w, blk, 0)
        plsc.subcore_barrier()

        def drain(i, _):
            off = r0 + i * DR
            pltpu.sync_copy(acc_sh.at[pl.ds(off, DR)],
                            out_hbm.at[c].at[pl.ds(off, DR)])
            return 0

        lax.fori_loop(0, nch, drain, 0)

    return k(dst, ew)


def _aggregate_partials(y, src, dst, ew):
    n, d = y.shape
    e = src.shape[0]
    nblk = e // B

    @functools.partial(
        pl.kernel,
        out_type=jax.ShapeDtypeStruct((NC, n, d), jnp.float32),
        mesh=_sc_mesh(),
        scratch_types=[
            pltpu.VMEM((2, B), jnp.int32),
            pltpu.VMEM((2, B), jnp.int32),
            pltpu.VMEM((2, B), jnp.float32),
            pltpu.VMEM((2, B, d), jnp.float32),
            pltpu.VMEM_SHARED((n, d), jnp.float32),
            pltpu.SemaphoreType.DMA,
            pltpu.SemaphoreType.DMA,
        ],
    )
    def k(y_hbm, src_hbm, dst_hbm, ew_hbm, out_hbm,
          sidx_v, didx_v, ew_v, rows_v, acc_sh, sem_i, sem_g):
        c = lax.axis_index("c")
        s = lax.axis_index("s")
        wid = s * NC + c

        zero16 = jnp.zeros((LANES,), jnp.float32)

        def zrow(i, _):
            for j in range(d // LANES):
                rows_v[0, i, pl.ds(j * LANES, LANES)] = zero16
            return 0

        lax.fori_loop(0, B, zrow, 0)

        r0 = s * TR
        nch = jnp.where(s < NS - 1, TR // DR, (n - (NS - 1) * TR) // DR)

        def zacc(i, _):
            pltpu.sync_copy(rows_v.at[0].at[pl.ds(0, DR)],
                            acc_sh.at[pl.ds(r0 + i * DR, DR)])
            return 0

        lax.fori_loop(0, nch, zacc, 0)
        plsc.subcore_barrier()

        nblk_w = (nblk // NW) + jnp.where(wid < (nblk % NW), 1, 0)

        def issue_idx(b, p):
            off = (wid + b * NW) * B
            pltpu.async_copy(src_hbm.at[pl.ds(off, B)], sidx_v.at[p], sem_i)
            pltpu.async_copy(dst_hbm.at[pl.ds(off, B)], didx_v.at[p], sem_i)
            pltpu.async_copy(ew_hbm.at[pl.ds(off, B)], ew_v.at[p], sem_i)

        def wait_idx(p):
            pltpu.make_async_copy(src_hbm.at[pl.ds(0, B)],
                                  sidx_v.at[p], sem_i).wait()
            pltpu.make_async_copy(dst_hbm.at[pl.ds(0, B)],
                                  didx_v.at[p], sem_i).wait()
            pltpu.make_async_copy(ew_hbm.at[pl.ds(0, B)],
                                  ew_v.at[p], sem_i).wait()

        def issue_gather(p):
            pltpu.async_copy(y_hbm.at[sidx_v.at[p]], rows_v.at[p], sem_g)

        def wait_gather(p):
            pltpu.make_async_copy(y_hbm.at[sidx_v.at[p]],
                                  rows_v.at[p], sem_g).wait()

        issue_idx(0, 0)
        wait_idx(0)
        issue_gather(0)
        issue_idx(1, 1)

        def blk(b, _):
            p = lax.rem(b, 2)
            q = lax.rem(b + 1, 2)
            wait_gather(p)

            @pl.when(b + 1 < nblk_w)
            def _():
                wait_idx(q)
                issue_gather(q)

            def sgrp(g, _):
                sv16 = ew_v[p, pl.ds(g * LANES, LANES)]
                for r in range(LANES):
                    sv = jnp.broadcast_to(sv16[r], (LANES,))
                    row = g * LANES + r
                    for j in range(d // LANES):
                        rows_v[p, row, pl.ds(j * LANES, LANES)] = (
                            rows_v[p, row, pl.ds(j * LANES, LANES)] * sv)
                return 0

            lax.fori_loop(0, B // LANES, sgrp, 0)
            pltpu.sync_copy(rows_v.at[p], acc_sh.at[didx_v.at[p]], add=True)

            @pl.when(b + 2 < nblk_w)
            def _():
                issue_idx(b + 2, p)

            return 0

        lax.fori_loop(0, nblk_w, blk, 0)
        plsc.subcore_barrier()

        def drain(i, _):
            off = r0 + i * DR
            pltpu.sync_copy(acc_sh.at[pl.ds(off, DR)],
                            out_hbm.at[c].at[pl.ds(off, DR)])
            return 0

        lax.fori_loop(0, nch, drain, 0)

    return k(y, src, dst, ew)


BM = 1000


def _dinv_block(degp_blk):
    deg = degp_blk[0, :, 0:1] + degp_blk[1, :, 0:1] + 1.0
    return lax.rsqrt(jnp.maximum(deg, 1e-12))


def _tc_pre(x, w1, degp):
    n, din = x.shape
    hid = w1.shape[1]

    def body(x_ref, w_ref, dp_ref, xw_ref, y_ref):
        dv = _dinv_block(dp_ref[...])
        xw = jnp.dot(x_ref[...], w_ref[...], preferred_element_type=jnp.float32)
        xw_ref[...] = xw
        y_ref[...] = xw * dv

    return pl.pallas_call(
        body,
        grid=(n // BM,),
        in_specs=[
            pl.BlockSpec((BM, din), lambda i: (i, 0)),
            pl.BlockSpec((din, hid), lambda i: (0, 0)),
            pl.BlockSpec((NC, BM, DW), lambda i: (0, i, 0)),
        ],
        out_specs=[
            pl.BlockSpec((BM, hid), lambda i: (i, 0)),
            pl.BlockSpec((BM, hid), lambda i: (i, 0)),
        ],
        out_shape=[
            jax.ShapeDtypeStruct((n, hid), jnp.float32),
            jax.ShapeDtypeStruct((n, hid), jnp.float32),
        ],
    )(x, w1, degp)


def _tc_mid(p, xw, degp, b, w2):
    n, hid = xw.shape
    hid2 = w2.shape[1]

    def body(p_ref, xw_ref, dp_ref, b_ref, w_ref, xw2_ref, y2_ref):
        dv = _dinv_block(dp_ref[...])
        agg = p_ref[0] + p_ref[1]
        h = jnp.maximum(dv * agg + (dv * dv) * xw_ref[...] + b_ref[...], 0.0)
        xw2 = jnp.dot(h, w_ref[...], preferred_element_type=jnp.float32)
        xw2_ref[...] = xw2
        y2_ref[...] = xw2 * dv

    return pl.pallas_call(
        body,
        grid=(n // BM,),
        in_specs=[
            pl.BlockSpec((NC, BM, hid), lambda i: (0, i, 0)),
            pl.BlockSpec((BM, hid), lambda i: (i, 0)),
            pl.BlockSpec((NC, BM, DW), lambda i: (0, i, 0)),
            pl.BlockSpec((1, hid), lambda i: (0, 0)),
            pl.BlockSpec((hid, hid2), lambda i: (0, 0)),
        ],
        out_specs=[
            pl.BlockSpec((BM, hid2), lambda i: (i, 0)),
            pl.BlockSpec((BM, hid2), lambda i: (i, 0)),
        ],
        out_shape=[
            jax.ShapeDtypeStruct((n, hid2), jnp.float32),
            jax.ShapeDtypeStruct((n, hid2), jnp.float32),
        ],
    )(p, xw, degp, b, w2)


def _tc_out(p, xw, degp, b, wfc, bfc):
    n, hid = xw.shape
    dout = wfc.shape[1]

    def body(p_ref, xw_ref, dp_ref, b_ref, w_ref, bfc_ref, out_ref):
        dv = _dinv_block(dp_ref[...])
        agg = p_ref[0] + p_ref[1]
        h = jnp.maximum(dv * agg + (dv * dv) * xw_ref[...] + b_ref[...], 0.0)
        out_ref[...] = (
            jnp.dot(h, w_ref[...], preferred_element_type=jnp.float32)
            + bfc_ref[...])

    return pl.pallas_call(
        body,
        grid=(n // BM,),
        in_specs=[
            pl.BlockSpec((NC, BM, hid), lambda i: (0, i, 0)),
            pl.BlockSpec((BM, hid), lambda i: (i, 0)),
            pl.BlockSpec((NC, BM, DW), lambda i: (0, i, 0)),
            pl.BlockSpec((1, hid), lambda i: (0, 0)),
            pl.BlockSpec((hid, dout), lambda i: (0, 0)),
            pl.BlockSpec((1, dout), lambda i: (0, 0)),
        ],
        out_specs=pl.BlockSpec((BM, dout), lambda i: (i, 0)),
        out_shape=jax.ShapeDtypeStruct((n, dout), jnp.float32),
    )(p, xw, degp, b, wfc, bfc)


def kernel(x, edge_index, edge_attr, W1, b1, W2, b2, Wfc, bfc):
    n = x.shape[0]
    src = edge_index[0].astype(jnp.int32)
    dst = edge_index[1].astype(jnp.int32)
    ew = edge_attr.astype(jnp.float32)

    degp = _degree_partials(dst, ew, n)

    xw1, y1 = _tc_pre(x, W1, degp)
    p1 = _aggregate_partials(y1, src, dst, ew)
    xw2, y2 = _tc_mid(p1, xw1, degp, b1.reshape(1, -1), W2)
    p2 = _aggregate_partials(y2, src, dst, ew)
    out = _tc_out(p2, xw2, degp, b2.reshape(1, -1), Wfc, bfc.reshape(1, -1))
    return out


# speedup vs baseline: 2.9538x; 2.9538x over previous
import functools

import jax
import jax.numpy as jnp
from jax import lax
from jax.experimental import pallas as pl
from jax.experimental.pallas import tpu as pltpu
from jax.experimental.pallas import tpu_sc as plsc

NC = 2
NS = 16
NW = NC * NS
LANES = 16

B = 128
DR = 80
TR = 640

DW = 128
TRD = 632
NPAD = NS * TRD

DDEPTH = 3
ADEPTH = 3


def _sc_mesh():
    return plsc.VectorSubcoreMesh(core_axis_name="c", subcore_axis_name="s")


def _degree_partials(dst, ew, n):
    e = dst.shape[0]
    nblk = e // B

    @functools.partial(
        pl.kernel,
        out_type=jax.ShapeDtypeStruct((NC, NPAD, DW), jnp.float32),
        mesh=_sc_mesh(),
        scratch_types=[
            pltpu.VMEM((DDEPTH, B), jnp.int32),
            pltpu.VMEM((DDEPTH, B), jnp.float32),
            pltpu.VMEM((DDEPTH, B, DW), jnp.float32),
            pltpu.VMEM_SHARED((NPAD, DW), jnp.float32),
            pltpu.SemaphoreType.DMA,
            pltpu.SemaphoreType.DMA,
        ],
    )
    def k(dst_hbm, ew_hbm, out_hbm, didx_v, ew_v, rows_v, acc_sh,
          sem_i, sem_s):
        c = lax.axis_index("c")
        s = lax.axis_index("s")
        wid = s * NC + c

        zero16 = jnp.zeros((LANES,), jnp.float32)

        def zrow(i, _):
            for j in range(DW // LANES):
                rows_v[0, i, pl.ds(j * LANES, LANES)] = zero16
            return 0

        lax.fori_loop(0, B, zrow, 0)

        r0 = s * TRD

        def zacc(i, _):
            pltpu.sync_copy(rows_v.at[0],
                            acc_sh.at[pl.ds(r0 + i * B, B)])
            return 0

        lax.fori_loop(0, TRD // B, zacc, 0)
        pltpu.sync_copy(rows_v.at[0].at[pl.ds(0, TRD - (TRD // B) * B)],
                        acc_sh.at[pl.ds(r0 + (TRD // B) * B,
                                        TRD - (TRD // B) * B)])
        plsc.subcore_barrier()

        nblk_w = (nblk // NW) + jnp.where(wid < (nblk % NW), 1, 0)

        def issue_idx(b, p):
            off = (wid + b * NW) * B
            pltpu.async_copy(dst_hbm.at[pl.ds(off, B)], didx_v.at[p], sem_i)
            pltpu.async_copy(ew_hbm.at[pl.ds(off, B)], ew_v.at[p], sem_i)

        def wait_idx(p):
            pltpu.make_async_copy(dst_hbm.at[pl.ds(0, B)],
                                  didx_v.at[p], sem_i).wait()
            pltpu.make_async_copy(ew_hbm.at[pl.ds(0, B)],
                                  ew_v.at[p], sem_i).wait()

        def issue_scatter(p):
            pltpu.async_copy(rows_v.at[p], acc_sh.at[didx_v.at[p]], sem_s,
                             add=True)

        def wait_scatter(p):
            pltpu.make_async_copy(rows_v.at[p], acc_sh.at[didx_v.at[p]],
                                  sem_s).wait()

        def blk_body(b, p):
            @pl.when(b >= 2)
            def _():
                wait_scatter((p + 1) % DDEPTH)

            wait_idx(p)

            @pl.when(b + 1 < nblk_w)
            def _():
                issue_idx(b + 1, (p + 1) % DDEPTH)

            def sgrp(g, _):
                sv16 = ew_v[p, pl.ds(g * LANES, LANES)]
                for r in range(LANES):
                    sv = jnp.broadcast_to(sv16[r], (LANES,))
                    for j in range(DW // LANES):
                        rows_v[p, g * LANES + r, pl.ds(j * LANES, LANES)] = sv
                return 0

            lax.fori_loop(0, B // LANES, sgrp, 0)
            issue_scatter(p)

        issue_idx(0, 0)

        def grp(t, _):
            for j in range(DDEPTH):
                blk_body(DDEPTH * t + j, j)
            return 0

        ngrp = nblk_w // DDEPTH
        lax.fori_loop(0, ngrp, grp, 0)
        rem = nblk_w - ngrp * DDEPTH
        for j in range(DDEPTH - 1):
            @pl.when(rem >= j + 1)
            def _():
                blk_body(ngrp * DDEPTH + j, j)

        wait_scatter(0)
        wait_scatter(1)
        plsc.subcore_barrier()

        pltpu.sync_copy(acc_sh.at[pl.ds(r0, TRD)],
                        out_hbm.at[c].at[pl.ds(r0, TRD)])

    return k(dst, ew)


def _aggregate_partials(y, src, dst, ew):
    n, d = y.shape
    e = src.shape[0]
    nblk = e // B

    @functools.partial(
        pl.kernel,
        out_type=jax.ShapeDtypeStruct((NC, n, d), jnp.float32),
        mesh=_sc_mesh(),
        scratch_types=[
            pltpu.VMEM((ADEPTH, B), jnp.int32),
            pltpu.VMEM((ADEPTH, B), jnp.int32),
            pltpu.VMEM((ADEPTH, B), jnp.float32),
            pltpu.VMEM((ADEPTH, B, d), jnp.float32),
            pltpu.VMEM_SHARED((n, d), jnp.float32),
            pltpu.SemaphoreType.DMA,
            pltpu.SemaphoreType.DMA,
            pltpu.SemaphoreType.DMA,
        ],
    )
    def k(y_hbm, src_hbm, dst_hbm, ew_hbm, out_hbm,
          sidx_v, didx_v, ew_v, rows_v, acc_sh, sem_i, sem_g, sem_s):
        c = lax.axis_index("c")
        s = lax.axis_index("s")
        wid = s * NC + c

        zero16 = jnp.zeros((LANES,), jnp.float32)

        def zrow(i, _):
            for j in range(d // LANES):
                rows_v[0, i, pl.ds(j * LANES, LANES)] = zero16
            return 0

        lax.fori_loop(0, B, zrow, 0)

        r0 = s * TR
        nch = jnp.where(s < NS - 1, TR // DR, (n - (NS - 1) * TR) // DR)

        def zacc(i, _):
            pltpu.sync_copy(rows_v.at[0].at[pl.ds(0, DR)],
                            acc_sh.at[pl.ds(r0 + i * DR, DR)])
            return 0

        lax.fori_loop(0, nch, zacc, 0)
        plsc.subcore_barrier()

        nblk_w = (nblk // NW) + jnp.where(wid < (nblk % NW), 1, 0)

        def issue_idx(b, p):
            off = (wid + b * NW) * B
            pltpu.async_copy(src_hbm.at[pl.ds(off, B)], sidx_v.at[p], sem_i)
            pltpu.async_copy(dst_hbm.at[pl.ds(off, B)], didx_v.at[p], sem_i)
            pltpu.async_copy(ew_hbm.at[pl.ds(off, B)], ew_v.at[p], sem_i)

        def wait_idx(p):
            pltpu.make_async_copy(src_hbm.at[pl.ds(0, B)],
                                  sidx_v.at[p], sem_i).wait()
            pltpu.make_async_copy(dst_hbm.at[pl.ds(0, B)],
                                  didx_v.at[p], sem_i).wait()
            pltpu.make_async_copy(ew_hbm.at[pl.ds(0, B)],
                                  ew_v.at[p], sem_i).wait()

        def issue_gather(p):
            pltpu.async_copy(y_hbm.at[sidx_v.at[p]], rows_v.at[p], sem_g)

        def wait_gather(p):
            pltpu.make_async_copy(y_hbm.at[sidx_v.at[p]],
                                  rows_v.at[p], sem_g).wait()

        def issue_scatter(p):
            pltpu.async_copy(rows_v.at[p], acc_sh.at[didx_v.at[p]], sem_s,
                             add=True)

        def wait_scatter(p):
            pltpu.make_async_copy(rows_v.at[p], acc_sh.at[didx_v.at[p]],
                                  sem_s).wait()

        def scale_rows(p):
            def sgrp(g, _):
                sv16 = ew_v[p, pl.ds(g * LANES, LANES)]
                for r in range(LANES):
                    sv = jnp.broadcast_to(sv16[r], (LANES,))
                    row = g * LANES + r
                    for j in range(d // LANES):
                        rows_v[p, row, pl.ds(j * LANES, LANES)] = (
                            rows_v[p, row, pl.ds(j * LANES, LANES)] * sv)
                return 0

            lax.fori_loop(0, B // LANES, sgrp, 0)

        def blk_body(b, p):
            @pl.when(b >= 1)
            def _():
                wait_scatter((p + 2) % ADEPTH)

            wait_gather(p)

            @pl.when(b + 1 < nblk_w)
            def _():
                wait_idx((p + 1) % ADEPTH)
                issue_gather((p + 1) % ADEPTH)

            scale_rows(p)
            issue_scatter(p)

            @pl.when(b + 2 < nblk_w)
            def _():
                issue_idx(b + 2, (p + 2) % ADEPTH)

        issue_idx(0, 0)
        wait_idx(0)
        issue_gather(0)
        issue_idx(1, 1)

        def grp(t, _):
            for j in range(ADEPTH):
                blk_body(ADEPTH * t + j, j)
            return 0

        ngrp = nblk_w // ADEPTH
        lax.fori_loop(0, ngrp, grp, 0)
        rem = nblk_w - ngrp * ADEPTH
        for j in range(ADEPTH - 1):
            @pl.when(rem >= j + 1)
            def _():
                blk_body(ngrp * ADEPTH + j, j)

        wait_scatter(0)
        plsc.subcore_barrier()

        def drain(i, _):
            off = r0 + i * DR
            pltpu.sync_copy(acc_sh.at[pl.ds(off, DR)],
                            out_hbm.at[c].at[pl.ds(off, DR)])
            return 0

        lax.fori_loop(0, nch, drain, 0)

    return k(y, src, dst, ew)


BM = 1000


def _dinv_block(degp_blk):
    deg = degp_blk[0, :, 0:1] + degp_blk[1, :, 0:1] + 1.0
    return lax.rsqrt(jnp.maximum(deg, 1e-12))


def _tc_pre(x, w1, degp):
    n, din = x.shape
    hid = w1.shape[1]

    def body(x_ref, w_ref, dp_ref, xw_ref, y_ref):
        dv = _dinv_block(dp_ref[...])
        xw = jnp.dot(x_ref[...], w_ref[...], preferred_element_type=jnp.float32)
        xw_ref[...] = xw
        y_ref[...] = xw * dv

    return pl.pallas_call(
        body,
        grid=(n // BM,),
        in_specs=[
            pl.BlockSpec((BM, din), lambda i: (i, 0)),
            pl.BlockSpec((din, hid), lambda i: (0, 0)),
            pl.BlockSpec((NC, BM, DW), lambda i: (0, i, 0)),
        ],
        out_specs=[
            pl.BlockSpec((BM, hid), lambda i: (i, 0)),
            pl.BlockSpec((BM, hid), lambda i: (i, 0)),
        ],
        out_shape=[
            jax.ShapeDtypeStruct((n, hid), jnp.float32),
            jax.ShapeDtypeStruct((n, hid), jnp.float32),
        ],
    )(x, w1, degp)


def _tc_mid(p, xw, degp, b, w2):
    n, hid = xw.shape
    hid2 = w2.shape[1]

    def body(p_ref, xw_ref, dp_ref, b_ref, w_ref, xw2_ref, y2_ref):
        dv = _dinv_block(dp_ref[...])
        agg = p_ref[0] + p_ref[1]
        h = jnp.maximum(dv * agg + (dv * dv) * xw_ref[...] + b_ref[...], 0.0)
        xw2 = jnp.dot(h, w_ref[...], preferred_element_type=jnp.float32)
        xw2_ref[...] = xw2
        y2_ref[...] = xw2 * dv

    return pl.pallas_call(
        body,
        grid=(n // BM,),
        in_specs=[
            pl.BlockSpec((NC, BM, hid), lambda i: (0, i, 0)),
            pl.BlockSpec((BM, hid), lambda i: (i, 0)),
            pl.BlockSpec((NC, BM, DW), lambda i: (0, i, 0)),
            pl.BlockSpec((1, hid), lambda i: (0, 0)),
            pl.BlockSpec((hid, hid2), lambda i: (0, 0)),
        ],
        out_specs=[
            pl.BlockSpec((BM, hid2), lambda i: (i, 0)),
            pl.BlockSpec((BM, hid2), lambda i: (i, 0)),
        ],
        out_shape=[
            jax.ShapeDtypeStruct((n, hid2), jnp.float32),
            jax.ShapeDtypeStruct((n, hid2), jnp.float32),
        ],
    )(p, xw, degp, b, w2)


def _tc_out(p, xw, degp, b, wfc, bfc):
    n, hid = xw.shape
    dout = wfc.shape[1]

    def body(p_ref, xw_ref, dp_ref, b_ref, w_ref, bfc_ref, out_ref):
        dv = _dinv_block(dp_ref[...])
        agg = p_ref[0] + p_ref[1]
        h = jnp.maximum(dv * agg + (dv * dv) * xw_ref[...] + b_ref[...], 0.0)
        out_ref[...] = (
            jnp.dot(h, w_ref[...], preferred_element_type=jnp.float32)
            + bfc_ref[...])

    return pl.pallas_call(
        body,
        grid=(n // BM,),
        in_specs=[
            pl.BlockSpec((NC, BM, hid), lambda i: (0, i, 0)),
            pl.BlockSpec((BM, hid), lambda i: (i, 0)),
            pl.BlockSpec((NC, BM, DW), lambda i: (0, i, 0)),
            pl.BlockSpec((1, hid), lambda i: (0, 0)),
            pl.BlockSpec((hid, dout), lambda i: (0, 0)),
            pl.BlockSpec((1, dout), lambda i: (0, 0)),
        ],
        out_specs=pl.BlockSpec((BM, dout), lambda i: (i, 0)),
        out_shape=jax.ShapeDtypeStruct((n, dout), jnp.float32),
    )(p, xw, degp, b, wfc, bfc)


def kernel(x, edge_index, edge_attr, W1, b1, W2, b2, Wfc, bfc):
    n = x.shape[0]
    src = edge_index[0].astype(jnp.int32)
    dst = edge_index[1].astype(jnp.int32)
    ew = edge_attr.astype(jnp.float32)

    degp = _degree_partials(dst, ew, n)

    xw1, y1 = _tc_pre(x, W1, degp)
    p1 = _aggregate_partials(y1, src, dst, ew)
    xw2, y2 = _tc_mid(p1, xw1, degp, b1.reshape(1, -1), W2)
    p2 = _aggregate_partials(y2, src, dst, ew)
    out = _tc_out(p2, xw2, degp, b2.reshape(1, -1), Wfc, bfc.reshape(1, -1))
    return out


# speedup vs baseline: 2.9675x; 1.0046x over previous
import functools

import jax
import jax.numpy as jnp
from jax import lax
from jax.experimental import pallas as pl
from jax.experimental.pallas import tpu as pltpu
from jax.experimental.pallas import tpu_sc as plsc

NC = 2
NS = 16
NW = NC * NS
LANES = 16

B = 128
DR = 80
TR = 640

DW = 128
TRD = 632
NPAD = NS * TRD

DDEPTH = 3
ADEPTH = 3


def _sc_mesh():
    return plsc.VectorSubcoreMesh(core_axis_name="c", subcore_axis_name="s")


def _degree_partials(dst, ew, n):
    e = dst.shape[0]
    nblk = e // B

    @functools.partial(
        pl.kernel,
        out_type=jax.ShapeDtypeStruct((NC, NPAD, DW), jnp.float32),
        mesh=_sc_mesh(),
        scratch_types=[
            pltpu.VMEM((DDEPTH, B), jnp.int32),
            pltpu.VMEM((DDEPTH, B), jnp.float32),
            pltpu.VMEM((DDEPTH, B, DW), jnp.float32),
            pltpu.VMEM_SHARED((NPAD, DW), jnp.float32),
            pltpu.SemaphoreType.DMA,
            pltpu.SemaphoreType.DMA,
        ],
    )
    def k(dst_hbm, ew_hbm, out_hbm, didx_v, ew_v, rows_v, acc_sh,
          sem_i, sem_s):
        c = lax.axis_index("c")
        s = lax.axis_index("s")
        wid = s * NC + c

        zero16 = jnp.zeros((LANES,), jnp.float32)

        def zrow(i, _):
            for p in range(DDEPTH):
                for j in range(DW // LANES):
                    rows_v[p, i, pl.ds(j * LANES, LANES)] = zero16
            return 0

        lax.fori_loop(0, B, zrow, 0)

        r0 = s * TRD

        def zacc(i, _):
            pltpu.sync_copy(rows_v.at[0],
                            acc_sh.at[pl.ds(r0 + i * B, B)])
            return 0

        lax.fori_loop(0, TRD // B, zacc, 0)
        pltpu.sync_copy(rows_v.at[0].at[pl.ds(0, TRD - (TRD // B) * B)],
                        acc_sh.at[pl.ds(r0 + (TRD // B) * B,
                                        TRD - (TRD // B) * B)])
        plsc.subcore_barrier()

        nblk_w = (nblk // NW) + jnp.where(wid < (nblk % NW), 1, 0)

        def issue_idx(b, p):
            off = (wid + b * NW) * B
            pltpu.async_copy(dst_hbm.at[pl.ds(off, B)], didx_v.at[p], sem_i)
            pltpu.async_copy(ew_hbm.at[pl.ds(off, B)], ew_v.at[p], sem_i)

        def wait_idx(p):
            pltpu.make_async_copy(dst_hbm.at[pl.ds(0, B)],
                                  didx_v.at[p], sem_i).wait()
            pltpu.make_async_copy(ew_hbm.at[pl.ds(0, B)],
                                  ew_v.at[p], sem_i).wait()

        def issue_scatter(p):
            pltpu.async_copy(rows_v.at[p], acc_sh.at[didx_v.at[p]], sem_s,
                             add=True)

        def wait_scatter(p):
            pltpu.make_async_copy(rows_v.at[p], acc_sh.at[didx_v.at[p]],
                                  sem_s).wait()

        def blk_body(b, p):
            @pl.when(b >= 2)
            def _():
                wait_scatter((p + 1) % DDEPTH)

            wait_idx(p)

            @pl.when(b + 1 < nblk_w)
            def _():
                issue_idx(b + 1, (p + 1) % DDEPTH)

            def sgrp(g, _):
                sv16 = ew_v[p, pl.ds(g * LANES, LANES)]
                for r in range(LANES):
                    rows_v[p, g * LANES + r, pl.ds(0, LANES)] = (
                        jnp.broadcast_to(sv16[r], (LANES,)))
                return 0

            lax.fori_loop(0, B // LANES, sgrp, 0)
            issue_scatter(p)

        issue_idx(0, 0)

        def grp(t, _):
            for j in range(DDEPTH):
                blk_body(DDEPTH * t + j, j)
            return 0

        ngrp = nblk_w // DDEPTH
        lax.fori_loop(0, ngrp, grp, 0)
        rem = nblk_w - ngrp * DDEPTH
        for j in range(DDEPTH - 1):
            @pl.when(rem >= j + 1)
            def _():
                blk_body(ngrp * DDEPTH + j, j)

        wait_scatter(0)
        wait_scatter(1)
        plsc.subcore_barrier()

        pltpu.sync_copy(acc_sh.at[pl.ds(r0, TRD)],
                        out_hbm.at[c].at[pl.ds(r0, TRD)])

    return k(dst, ew)


def _aggregate_partials(y, src, dst, ew):
    n, d = y.shape
    e = src.shape[0]
    nblk = e // B

    @functools.partial(
        pl.kernel,
        out_type=jax.ShapeDtypeStruct((NC, n, d), jnp.float32),
        mesh=_sc_mesh(),
        scratch_types=[
            pltpu.VMEM((ADEPTH, B), jnp.int32),
            pltpu.VMEM((ADEPTH, B), jnp.int32),
            pltpu.VMEM((ADEPTH, B), jnp.float32),
            pltpu.VMEM((ADEPTH, B, d), jnp.float32),
            pltpu.VMEM_SHARED((n, d), jnp.float32),
            pltpu.SemaphoreType.DMA,
            pltpu.SemaphoreType.DMA,
            pltpu.SemaphoreType.DMA,
        ],
    )
    def k(y_hbm, src_hbm, dst_hbm, ew_hbm, out_hbm,
          sidx_v, didx_v, ew_v, rows_v, acc_sh, sem_i, sem_g, sem_s):
        c = lax.axis_index("c")
        s = lax.axis_index("s")
        wid = s * NC + c

        zero16 = jnp.zeros((LANES,), jnp.float32)

        def zrow(i, _):
            for j in range(d // LANES):
                rows_v[0, i, pl.ds(j * LANES, LANES)] = zero16
            return 0

        lax.fori_loop(0, B, zrow, 0)

        r0 = s * TR
        nch = jnp.where(s < NS - 1, TR // DR, (n - (NS - 1) * TR) // DR)

        def zacc(i, _):
            pltpu.sync_copy(rows_v.at[0].at[pl.ds(0, DR)],
                            acc_sh.at[pl.ds(r0 + i * DR, DR)])
            return 0

        lax.fori_loop(0, nch, zacc, 0)
        plsc.subcore_barrier()

        nblk_w = (nblk // NW) + jnp.where(wid < (nblk % NW), 1, 0)

        def issue_idx(b, p):
            off = (wid + b * NW) * B
            pltpu.async_copy(src_hbm.at[pl.ds(off, B)], sidx_v.at[p], sem_i)
            pltpu.async_copy(dst_hbm.at[pl.ds(off, B)], didx_v.at[p], sem_i)
            pltpu.async_copy(ew_hbm.at[pl.ds(off, B)], ew_v.at[p], sem_i)

        def wait_idx(p):
            pltpu.make_async_copy(src_hbm.at[pl.ds(0, B)],
                                  sidx_v.at[p], sem_i).wait()
            pltpu.make_async_copy(dst_hbm.at[pl.ds(0, B)],
                                  didx_v.at[p], sem_i).wait()
            pltpu.make_async_copy(ew_hbm.at[pl.ds(0, B)],
                                  ew_v.at[p], sem_i).wait()

        def issue_gather(p):
            pltpu.async_copy(y_hbm.at[sidx_v.at[p]], rows_v.at[p], sem_g)

        def wait_gather(p):
            pltpu.make_async_copy(y_hbm.at[sidx_v.at[p]],
                                  rows_v.at[p], sem_g).wait()

        def issue_scatter(p):
            pltpu.async_copy(rows_v.at[p], acc_sh.at[didx_v.at[p]], sem_s,
                             add=True)

        def wait_scatter(p):
            pltpu.make_async_copy(rows_v.at[p], acc_sh.at[didx_v.at[p]],
                                  sem_s).wait()

        def scale_rows(p):
            def sgrp(g, _):
                sv16 = ew_v[p, pl.ds(g * LANES, LANES)]
                for r in range(LANES):
                    sv = jnp.broadcast_to(sv16[r], (LANES,))
                    row = g * LANES + r
                    for j in range(d // LANES):
                        rows_v[p, row, pl.ds(j * LANES, LANES)] = (
                            rows_v[p, row, pl.ds(j * LANES, LANES)] * sv)
                return 0

            lax.fori_loop(0, B // LANES, sgrp, 0)

        def blk_body(b, p):
            @pl.when(b >= 1)
            def _():
                wait_scatter((p + 2) % ADEPTH)

            wait_gather(p)

            @pl.when(b + 1 < nblk_w)
            def _():
                wait_idx((p + 1) % ADEPTH)
                issue_gather((p + 1) % ADEPTH)

            scale_rows(p)
            issue_scatter(p)

            @pl.when(b + 2 < nblk_w)
            def _():
                issue_idx(b + 2, (p + 2) % ADEPTH)

        issue_idx(0, 0)
        wait_idx(0)
        issue_gather(0)
        issue_idx(1, 1)

        def grp(t, _):
            for j in range(ADEPTH):
                blk_body(ADEPTH * t + j, j)
            return 0

        ngrp = nblk_w // ADEPTH
        lax.fori_loop(0, ngrp, grp, 0)
        rem = nblk_w - ngrp * ADEPTH
        for j in range(ADEPTH - 1):
            @pl.when(rem >= j + 1)
            def _():
                blk_body(ngrp * ADEPTH + j, j)

        wait_scatter(0)
        plsc.subcore_barrier()

        def drain(i, _):
            off = r0 + i * DR
            pltpu.sync_copy(acc_sh.at[pl.ds(off, DR)],
                            out_hbm.at[c].at[pl.ds(off, DR)])
            return 0

        lax.fori_loop(0, nch, drain, 0)

    return k(y, src, dst, ew)


BM = 1000


def _dinv_block(degp_blk):
    deg = degp_blk[0, :, 0:1] + degp_blk[1, :, 0:1] + 1.0
    return lax.rsqrt(jnp.maximum(deg, 1e-12))


def _tc_pre(x, w1, degp):
    n, din = x.shape
    hid = w1.shape[1]

    def body(x_ref, w_ref, dp_ref, xw_ref, y_ref):
        dv = _dinv_block(dp_ref[...])
        xw = jnp.dot(x_ref[...], w_ref[...], preferred_element_type=jnp.float32)
        xw_ref[...] = xw
        y_ref[...] = xw * dv

    return pl.pallas_call(
        body,
        grid=(n // BM,),
        in_specs=[
            pl.BlockSpec((BM, din), lambda i: (i, 0)),
            pl.BlockSpec((din, hid), lambda i: (0, 0)),
            pl.BlockSpec((NC, BM, DW), lambda i: (0, i, 0)),
        ],
        out_specs=[
            pl.BlockSpec((BM, hid), lambda i: (i, 0)),
            pl.BlockSpec((BM, hid), lambda i: (i, 0)),
        ],
        out_shape=[
            jax.ShapeDtypeStruct((n, hid), jnp.float32),
            jax.ShapeDtypeStruct((n, hid), jnp.float32),
        ],
    )(x, w1, degp)


def _tc_mid(p, xw, degp, b, w2):
    n, hid = xw.shape
    hid2 = w2.shape[1]

    def body(p_ref, xw_ref, dp_ref, b_ref, w_ref, xw2_ref, y2_ref):
        dv = _dinv_block(dp_ref[...])
        agg = p_ref[0] + p_ref[1]
        h = jnp.maximum(dv * agg + (dv * dv) * xw_ref[...] + b_ref[...], 0.0)
        xw2 = jnp.dot(h, w_ref[...], preferred_element_type=jnp.float32)
        xw2_ref[...] = xw2
        y2_ref[...] = xw2 * dv

    return pl.pallas_call(
        body,
        grid=(n // BM,),
        in_specs=[
            pl.BlockSpec((NC, BM, hid), lambda i: (0, i, 0)),
            pl.BlockSpec((BM, hid), lambda i: (i, 0)),
            pl.BlockSpec((NC, BM, DW), lambda i: (0, i, 0)),
            pl.BlockSpec((1, hid), lambda i: (0, 0)),
            pl.BlockSpec((hid, hid2), lambda i: (0, 0)),
        ],
        out_specs=[
            pl.BlockSpec((BM, hid2), lambda i: (i, 0)),
            pl.BlockSpec((BM, hid2), lambda i: (i, 0)),
        ],
        out_shape=[
            jax.ShapeDtypeStruct((n, hid2), jnp.float32),
            jax.ShapeDtypeStruct((n, hid2), jnp.float32),
        ],
    )(p, xw, degp, b, w2)


def _tc_out(p, xw, degp, b, wfc, bfc):
    n, hid = xw.shape
    dout = wfc.shape[1]

    def body(p_ref, xw_ref, dp_ref, b_ref, w_ref, bfc_ref, out_ref):
        dv = _dinv_block(dp_ref[...])
        agg = p_ref[0] + p_ref[1]
        h = jnp.maximum(dv * agg + (dv * dv) * xw_ref[...] + b_ref[...], 0.0)
        out_ref[...] = (
            jnp.dot(h, w_ref[...], preferred_element_type=jnp.float32)
            + bfc_ref[...])

    return pl.pallas_call(
        body,
        grid=(n // BM,),
        in_specs=[
            pl.BlockSpec((NC, BM, hid), lambda i: (0, i, 0)),
            pl.BlockSpec((BM, hid), lambda i: (i, 0)),
            pl.BlockSpec((NC, BM, DW), lambda i: (0, i, 0)),
            pl.BlockSpec((1, hid), lambda i: (0, 0)),
            pl.BlockSpec((hid, dout), lambda i: (0, 0)),
            pl.BlockSpec((1, dout), lambda i: (0, 0)),
        ],
        out_specs=pl.BlockSpec((BM, dout), lambda i: (i, 0)),
        out_shape=jax.ShapeDtypeStruct((n, dout), jnp.float32),
    )(p, xw, degp, b, wfc, bfc)


def kernel(x, edge_index, edge_attr, W1, b1, W2, b2, Wfc, bfc):
    n = x.shape[0]
    src = edge_index[0].astype(jnp.int32)
    dst = edge_index[1].astype(jnp.int32)
    ew = edge_attr.astype(jnp.float32)

    degp = _degree_partials(dst, ew, n)

    xw1, y1 = _tc_pre(x, W1, degp)
    p1 = _aggregate_partials(y1, src, dst, ew)
    xw2, y2 = _tc_mid(p1, xw1, degp, b1.reshape(1, -1), W2)
    p2 = _aggregate_partials(y2, src, dst, ew)
    out = _tc_out(p2, xw2, degp, b2.reshape(1, -1), Wfc, bfc.reshape(1, -1))
    return out
